# Initial kernel scaffold; baseline (speedup 1.0000x reference)
import jax
import jax.numpy as jnp
from jax.experimental import pallas as pl


def kernel(text_embeds, object_embeds, image_token_mask, num_objects, mlp1_ln_g, mlp1_ln_b, mlp1_fc1_w, mlp1_fc1_b, mlp1_fc2_w, mlp1_fc2_b, mlp2_ln_g, mlp2_ln_b, mlp2_fc1_w, mlp2_fc1_b, mlp2_fc2_w, mlp2_fc2_b, ln_g, ln_b):
    raise NotImplementedError("write your pallas kernel here")


# speedup vs baseline: 2.6451x; 2.6451x over previous
import jax
import jax.numpy as jnp
from jax.experimental import pallas as pl
from jax.experimental.pallas import tpu as pltpu

D = 768
TILE = 512

_INV_SQRT2 = 0.7071067811865476


def _gelu_exact(x):
    return 0.5 * x * (1.0 + jax.lax.erf(x * _INV_SQRT2))


def _body(t_ref, o_ref, m_ref, osc_ref,
          w1_ref, b1_ref, w2_ref, b2_ref, w3_ref, b3_ref, w4_ref, b4_ref,
          g1_ref, be1_ref, g2_ref, be2_ref, gf_ref, bf_ref,
          out_ref):
    t = t_ref[...]
    osc = osc_ref[0, 0, :][:, None]
    o = o_ref[...] * osc

    s1 = jnp.sum(t, axis=1, keepdims=True) + jnp.sum(o, axis=1, keepdims=True)
    s2 = (jnp.sum(t * t, axis=1, keepdims=True)
          + jnp.sum(o * o, axis=1, keepdims=True))
    mu = s1 / (2 * D)
    var = s2 / (2 * D) - mu * mu
    inv = jax.lax.rsqrt(var + 1e-5)
    tn = (t - mu) * inv * g1_ref[0, :D] + be1_ref[0, :D]
    on = (o - mu) * inv * g1_ref[0, D:] + be1_ref[0, D:]

    h = (jnp.dot(tn, w1_ref[:D, :], preferred_element_type=jnp.float32)
         + jnp.dot(on, w1_ref[D:, :], preferred_element_type=jnp.float32)
         + b1_ref[0, :])
    h = _gelu_exact(h)
    x1 = (jnp.dot(h, w2_ref[...], preferred_element_type=jnp.float32)
          + b2_ref[0, :] + t)

    mu2 = jnp.mean(x1, axis=1, keepdims=True)
    var2 = jnp.mean(x1 * x1, axis=1, keepdims=True) - mu2 * mu2
    ln2 = (x1 - mu2) * jax.lax.rsqrt(var2 + 1e-5) * g2_ref[0, :] + be2_ref[0, :]
    h2 = (jnp.dot(ln2, w3_ref[...], preferred_element_type=jnp.float32)
          + b3_ref[0, :])
    h2 = _gelu_exact(h2)
    x2 = (jnp.dot(h2, w4_ref[...], preferred_element_type=jnp.float32)
          + b4_ref[0, :] + x1)

    mu3 = jnp.mean(x2, axis=1, keepdims=True)
    var3 = jnp.mean(x2 * x2, axis=1, keepdims=True) - mu3 * mu3
    y = (x2 - mu3) * jax.lax.rsqrt(var3 + 1e-5) * gf_ref[0, :] + bf_ref[0, :]

    m = m_ref[0, 0, :][:, None]
    out_ref[...] = jnp.where(m > 0, y, t)


def kernel(text_embeds, object_embeds, image_token_mask, num_objects,
           mlp1_ln_g, mlp1_ln_b, mlp1_fc1_w, mlp1_fc1_b, mlp1_fc2_w, mlp1_fc2_b,
           mlp2_ln_g, mlp2_ln_b, mlp2_fc1_w, mlp2_fc1_b, mlp2_fc2_w, mlp2_fc2_b,
           ln_g, ln_b):
    B, S, _ = text_embeds.shape
    N = B * S
    nb = N // TILE

    t = text_embeds.reshape(N, D)
    o = object_embeds.reshape(N, D)
    m = image_token_mask.reshape(nb, 1, TILE).astype(jnp.float32)
    osc = jnp.repeat((num_objects > 0).astype(jnp.float32), S).reshape(nb, 1, TILE)

    w1 = mlp1_fc1_w.T
    w2 = mlp1_fc2_w.T
    w3 = mlp2_fc1_w.T
    w4 = mlp2_fc2_w.T
    row = lambda v: v.reshape(1, -1)

    full = lambda shape: pl.BlockSpec(shape, lambda i: (0,) * len(shape))
    out = pl.pallas_call(
        _body,
        grid=(nb,),
        in_specs=[
            pl.BlockSpec((TILE, D), lambda i: (i, 0)),
            pl.BlockSpec((TILE, D), lambda i: (i, 0)),
            pl.BlockSpec((1, 1, TILE), lambda i: (i, 0, 0)),
            pl.BlockSpec((1, 1, TILE), lambda i: (i, 0, 0)),
            full((2 * D, D)), full((1, D)),
            full((D, D)), full((1, D)),
            full((D, D)), full((1, D)),
            full((D, D)), full((1, D)),
            full((1, 2 * D)), full((1, 2 * D)),
            full((1, D)), full((1, D)),
            full((1, D)), full((1, D)),
        ],
        out_specs=pl.BlockSpec((TILE, D), lambda i: (i, 0)),
        out_shape=jax.ShapeDtypeStruct((N, D), jnp.float32),
        compiler_params=pltpu.CompilerParams(
            dimension_semantics=("arbitrary",)),
    )(t, o, m, osc,
      w1, row(mlp1_fc1_b), w2, row(mlp1_fc2_b),
      w3, row(mlp2_fc1_b), w4, row(mlp2_fc2_b),
      row(mlp1_ln_g), row(mlp1_ln_b), row(mlp2_ln_g), row(mlp2_ln_b),
      row(ln_g), row(ln_b))
    return out.reshape(B, S, D)


# speedup vs baseline: 2.6797x; 1.0131x over previous
import jax
import jax.numpy as jnp
from jax.experimental import pallas as pl
from jax.experimental.pallas import tpu as pltpu

D = 768
TILE = 512

_INV_SQRT2 = 0.7071067811865476


def _gelu_exact(x):
    return 0.5 * x * (1.0 + jax.lax.erf(x * _INV_SQRT2))


def _body(t_ref, o_ref, m_ref, osc_ref,
          w1_ref, b1_ref, w2_ref, b2_ref, w3_ref, b3_ref, w4_ref, b4_ref,
          g1_ref, be1_ref, g2_ref, be2_ref, gf_ref, bf_ref,
          out_ref):
    t = t_ref[...]
    osc = osc_ref[0, 0, :][:, None]
    o = o_ref[...] * osc

    s1 = jnp.sum(t, axis=1, keepdims=True) + jnp.sum(o, axis=1, keepdims=True)
    s2 = (jnp.sum(t * t, axis=1, keepdims=True)
          + jnp.sum(o * o, axis=1, keepdims=True))
    mu = s1 / (2 * D)
    var = s2 / (2 * D) - mu * mu
    inv = jax.lax.rsqrt(var + 1e-5)
    tn = (t - mu) * inv * g1_ref[0, :D] + be1_ref[0, :D]
    on = (o - mu) * inv * g1_ref[0, D:] + be1_ref[0, D:]

    bf = jnp.bfloat16
    h = (jnp.dot(tn.astype(bf), w1_ref[:D, :], preferred_element_type=jnp.float32)
         + jnp.dot(on.astype(bf), w1_ref[D:, :], preferred_element_type=jnp.float32)
         + b1_ref[0, :])
    h = _gelu_exact(h)
    x1 = (jnp.dot(h.astype(bf), w2_ref[...], preferred_element_type=jnp.float32)
          + b2_ref[0, :] + t)

    mu2 = jnp.mean(x1, axis=1, keepdims=True)
    var2 = jnp.mean(x1 * x1, axis=1, keepdims=True) - mu2 * mu2
    ln2 = (x1 - mu2) * jax.lax.rsqrt(var2 + 1e-5) * g2_ref[0, :] + be2_ref[0, :]
    h2 = (jnp.dot(ln2.astype(bf), w3_ref[...], preferred_element_type=jnp.float32)
          + b3_ref[0, :])
    h2 = _gelu_exact(h2)
    x2 = (jnp.dot(h2.astype(bf), w4_ref[...], preferred_element_type=jnp.float32)
          + b4_ref[0, :] + x1)

    mu3 = jnp.mean(x2, axis=1, keepdims=True)
    var3 = jnp.mean(x2 * x2, axis=1, keepdims=True) - mu3 * mu3
    y = (x2 - mu3) * jax.lax.rsqrt(var3 + 1e-5) * gf_ref[0, :] + bf_ref[0, :]

    m = m_ref[0, 0, :][:, None]
    out_ref[...] = jnp.where(m > 0, y, t)


def kernel(text_embeds, object_embeds, image_token_mask, num_objects,
           mlp1_ln_g, mlp1_ln_b, mlp1_fc1_w, mlp1_fc1_b, mlp1_fc2_w, mlp1_fc2_b,
           mlp2_ln_g, mlp2_ln_b, mlp2_fc1_w, mlp2_fc1_b, mlp2_fc2_w, mlp2_fc2_b,
           ln_g, ln_b):
    B, S, _ = text_embeds.shape
    N = B * S
    nb = N // TILE

    t = text_embeds.reshape(N, D)
    o = object_embeds.reshape(N, D)
    m = image_token_mask.reshape(nb, 1, TILE).astype(jnp.float32)
    osc = jnp.repeat((num_objects > 0).astype(jnp.float32), S).reshape(nb, 1, TILE)

    w1 = mlp1_fc1_w.T.astype(jnp.bfloat16)
    w2 = mlp1_fc2_w.T.astype(jnp.bfloat16)
    w3 = mlp2_fc1_w.T.astype(jnp.bfloat16)
    w4 = mlp2_fc2_w.T.astype(jnp.bfloat16)
    row = lambda v: v.reshape(1, -1)

    full = lambda shape: pl.BlockSpec(shape, lambda i: (0,) * len(shape))
    out = pl.pallas_call(
        _body,
        grid=(nb,),
        in_specs=[
            pl.BlockSpec((TILE, D), lambda i: (i, 0)),
            pl.BlockSpec((TILE, D), lambda i: (i, 0)),
            pl.BlockSpec((1, 1, TILE), lambda i: (i, 0, 0)),
            pl.BlockSpec((1, 1, TILE), lambda i: (i, 0, 0)),
            full((2 * D, D)), full((1, D)),
            full((D, D)), full((1, D)),
            full((D, D)), full((1, D)),
            full((D, D)), full((1, D)),
            full((1, 2 * D)), full((1, 2 * D)),
            full((1, D)), full((1, D)),
            full((1, D)), full((1, D)),
        ],
        out_specs=pl.BlockSpec((TILE, D), lambda i: (i, 0)),
        out_shape=jax.ShapeDtypeStruct((N, D), jnp.float32),
        compiler_params=pltpu.CompilerParams(
            dimension_semantics=("arbitrary",)),
    )(t, o, m, osc,
      w1, row(mlp1_fc1_b), w2, row(mlp1_fc2_b),
      w3, row(mlp2_fc1_b), w4, row(mlp2_fc2_b),
      row(mlp1_ln_g), row(mlp1_ln_b), row(mlp2_ln_g), row(mlp2_ln_b),
      row(ln_g), row(ln_b))
    return out.reshape(B, S, D)


# speedup vs baseline: 2.9584x; 1.1040x over previous
import jax
import jax.numpy as jnp
from jax.experimental import pallas as pl
from jax.experimental.pallas import tpu as pltpu

D = 768
TILE = 512

_INV_SQRT2 = 0.7071067811865476


def _gelu_exact(x):
    return 0.5 * x * (1.0 + jax.lax.erf(x * _INV_SQRT2))


def _body(t_ref, o_ref, m_ref, osc_ref,
          w1t_ref, w1o_ref, v1_ref, u1_ref, w2_ref, c2_ref,
          w3_ref, v3_ref, u3_ref, w4_ref, c4_ref,
          gf_ref, bf_ref,
          out_ref):
    bf = jnp.bfloat16
    t = t_ref[...]
    o = o_ref[...]
    osc = osc_ref[0, 0, :][:, None]

    st = jnp.sum(t, axis=1, keepdims=True)
    qt = jnp.sum(t * t, axis=1, keepdims=True)
    so = jnp.sum(o, axis=1, keepdims=True)
    qo = jnp.sum(o * o, axis=1, keepdims=True)
    mu = (st + osc * so) / (2 * D)
    var = (qt + osc * osc * qo) / (2 * D) - mu * mu
    inv = jax.lax.rsqrt(var + 1e-5)

    p = (jnp.dot(t.astype(bf), w1t_ref[...], preferred_element_type=jnp.float32)
         + osc * jnp.dot(o.astype(bf), w1o_ref[...],
                         preferred_element_type=jnp.float32))
    h = inv * p - (inv * mu) * v1_ref[0, :] + u1_ref[0, :]
    h = _gelu_exact(h)
    x1 = (jnp.dot(h.astype(bf), w2_ref[...], preferred_element_type=jnp.float32)
          + c2_ref[0, :] + t)

    s1 = jnp.sum(x1, axis=1, keepdims=True)
    q1 = jnp.sum(x1 * x1, axis=1, keepdims=True)
    mu2 = s1 / D
    inv2 = jax.lax.rsqrt(q1 / D - mu2 * mu2 + 1e-5)
    h2 = (inv2 * jnp.dot(x1.astype(bf), w3_ref[...],
                         preferred_element_type=jnp.float32)
          - (inv2 * mu2) * v3_ref[0, :] + u3_ref[0, :])
    h2 = _gelu_exact(h2)
    x2 = (jnp.dot(h2.astype(bf), w4_ref[...], preferred_element_type=jnp.float32)
          + c4_ref[0, :] + x1)

    mu3 = jnp.mean(x2, axis=1, keepdims=True)
    var3 = jnp.mean(x2 * x2, axis=1, keepdims=True) - mu3 * mu3
    y = (x2 - mu3) * jax.lax.rsqrt(var3 + 1e-5) * gf_ref[0, :] + bf_ref[0, :]

    m = m_ref[0, 0, :][:, None]
    out_ref[...] = jnp.where(m > 0, y, t)


def kernel(text_embeds, object_embeds, image_token_mask, num_objects,
           mlp1_ln_g, mlp1_ln_b, mlp1_fc1_w, mlp1_fc1_b, mlp1_fc2_w, mlp1_fc2_b,
           mlp2_ln_g, mlp2_ln_b, mlp2_fc1_w, mlp2_fc1_b, mlp2_fc2_w, mlp2_fc2_b,
           ln_g, ln_b):
    B, S, _ = text_embeds.shape
    N = B * S
    nb = N // TILE
    bf = jnp.bfloat16

    t = text_embeds.reshape(N, D)
    o = object_embeds.reshape(N, D)
    m = image_token_mask.reshape(nb, 1, TILE).astype(jnp.float32)
    osc = jnp.repeat((num_objects > 0).astype(jnp.float32), S).reshape(nb, 1, TILE)

    w1 = mlp1_fc1_w.T
    w1g = mlp1_ln_g[:, None] * w1
    w1t = w1g[:D, :].astype(bf)
    w1o = w1g[D:, :].astype(bf)
    v1 = (mlp1_ln_g @ w1).reshape(1, D)
    u1 = (mlp1_ln_b @ w1 + mlp1_fc1_b).reshape(1, D)
    w2 = mlp1_fc2_w.T.astype(bf)
    c2 = mlp1_fc2_b.reshape(1, D)
    w3 = mlp2_fc1_w.T
    w3g = (mlp2_ln_g[:, None] * w3).astype(bf)
    v3 = (mlp2_ln_g @ w3).reshape(1, D)
    u3 = (mlp2_ln_b @ w3 + mlp2_fc1_b).reshape(1, D)
    w4 = mlp2_fc2_w.T.astype(bf)
    c4 = mlp2_fc2_b.reshape(1, D)

    full = lambda shape: pl.BlockSpec(shape, lambda i: (0,) * len(shape))
    out = pl.pallas_call(
        _body,
        grid=(nb,),
        in_specs=[
            pl.BlockSpec((TILE, D), lambda i: (i, 0)),
            pl.BlockSpec((TILE, D), lambda i: (i, 0)),
            pl.BlockSpec((1, 1, TILE), lambda i: (i, 0, 0)),
            pl.BlockSpec((1, 1, TILE), lambda i: (i, 0, 0)),
            full((D, D)), full((D, D)), full((1, D)), full((1, D)),
            full((D, D)), full((1, D)),
            full((D, D)), full((1, D)), full((1, D)),
            full((D, D)), full((1, D)),
            full((1, D)), full((1, D)),
        ],
        out_specs=pl.BlockSpec((TILE, D), lambda i: (i, 0)),
        out_shape=jax.ShapeDtypeStruct((N, D), jnp.float32),
        compiler_params=pltpu.CompilerParams(
            dimension_semantics=("arbitrary",)),
    )(t, o, m, osc,
      w1t, w1o, v1, u1, w2, c2,
      w3g, v3, u3, w4, c4,
      ln_g.reshape(1, D), ln_b.reshape(1, D))
    return out.reshape(B, S, D)


# speedup vs baseline: 2.9653x; 1.0023x over previous
import jax
import jax.numpy as jnp
from jax.experimental import pallas as pl
from jax.experimental.pallas import tpu as pltpu

D = 768
TILE = 1024

_INV_SQRT2 = 0.7071067811865476


def _gelu_exact(x):
    return 0.5 * x * (1.0 + jax.lax.erf(x * _INV_SQRT2))


def _body(t_ref, o_ref, m_ref, osc_ref,
          w1t_ref, w1o_ref, v1_ref, u1_ref, w2_ref, c2_ref,
          w3_ref, v3_ref, u3_ref, w4_ref, c4_ref,
          gf_ref, bf_ref,
          out_ref):
    bf = jnp.bfloat16
    t = t_ref[...]
    o = o_ref[...]
    osc = osc_ref[0, 0, :][:, None]

    st = jnp.sum(t, axis=1, keepdims=True)
    qt = jnp.sum(t * t, axis=1, keepdims=True)
    so = jnp.sum(o, axis=1, keepdims=True)
    qo = jnp.sum(o * o, axis=1, keepdims=True)
    mu = (st + osc * so) / (2 * D)
    var = (qt + osc * osc * qo) / (2 * D) - mu * mu
    inv = jax.lax.rsqrt(var + 1e-5)

    p = (jnp.dot(t.astype(bf), w1t_ref[...], preferred_element_type=jnp.float32)
         + osc * jnp.dot(o.astype(bf), w1o_ref[...],
                         preferred_element_type=jnp.float32))
    h = inv * p - (inv * mu) * v1_ref[0, :] + u1_ref[0, :]
    h = _gelu_exact(h)
    x1 = (jnp.dot(h.astype(bf), w2_ref[...], preferred_element_type=jnp.float32)
          + c2_ref[0, :] + t)

    s1 = jnp.sum(x1, axis=1, keepdims=True)
    q1 = jnp.sum(x1 * x1, axis=1, keepdims=True)
    mu2 = s1 / D
    inv2 = jax.lax.rsqrt(q1 / D - mu2 * mu2 + 1e-5)
    h2 = (inv2 * jnp.dot(x1.astype(bf), w3_ref[...],
                         preferred_element_type=jnp.float32)
          - (inv2 * mu2) * v3_ref[0, :] + u3_ref[0, :])
    h2 = _gelu_exact(h2)
    x2 = (jnp.dot(h2.astype(bf), w4_ref[...], preferred_element_type=jnp.float32)
          + c4_ref[0, :] + x1)

    mu3 = jnp.mean(x2, axis=1, keepdims=True)
    var3 = jnp.mean(x2 * x2, axis=1, keepdims=True) - mu3 * mu3
    y = (x2 - mu3) * jax.lax.rsqrt(var3 + 1e-5) * gf_ref[0, :] + bf_ref[0, :]

    m = m_ref[0, 0, :][:, None]
    out_ref[...] = jnp.where(m > 0, y, t)


def kernel(text_embeds, object_embeds, image_token_mask, num_objects,
           mlp1_ln_g, mlp1_ln_b, mlp1_fc1_w, mlp1_fc1_b, mlp1_fc2_w, mlp1_fc2_b,
           mlp2_ln_g, mlp2_ln_b, mlp2_fc1_w, mlp2_fc1_b, mlp2_fc2_w, mlp2_fc2_b,
           ln_g, ln_b):
    B, S, _ = text_embeds.shape
    N = B * S
    nb = N // TILE
    bf = jnp.bfloat16

    t = text_embeds.reshape(N, D)
    o = object_embeds.reshape(N, D)
    m = image_token_mask.reshape(nb, 1, TILE).astype(jnp.float32)
    osc = jnp.repeat((num_objects > 0).astype(jnp.float32), S).reshape(nb, 1, TILE)

    w1 = mlp1_fc1_w.T
    w1g = mlp1_ln_g[:, None] * w1
    w1t = w1g[:D, :].astype(bf)
    w1o = w1g[D:, :].astype(bf)
    v1 = (mlp1_ln_g @ w1).reshape(1, D)
    u1 = (mlp1_ln_b @ w1 + mlp1_fc1_b).reshape(1, D)
    w2 = mlp1_fc2_w.T.astype(bf)
    c2 = mlp1_fc2_b.reshape(1, D)
    w3 = mlp2_fc1_w.T
    w3g = (mlp2_ln_g[:, None] * w3).astype(bf)
    v3 = (mlp2_ln_g @ w3).reshape(1, D)
    u3 = (mlp2_ln_b @ w3 + mlp2_fc1_b).reshape(1, D)
    w4 = mlp2_fc2_w.T.astype(bf)
    c4 = mlp2_fc2_b.reshape(1, D)

    full = lambda shape: pl.BlockSpec(shape, lambda i: (0,) * len(shape))
    out = pl.pallas_call(
        _body,
        grid=(nb,),
        in_specs=[
            pl.BlockSpec((TILE, D), lambda i: (i, 0)),
            pl.BlockSpec((TILE, D), lambda i: (i, 0)),
            pl.BlockSpec((1, 1, TILE), lambda i: (i, 0, 0)),
            pl.BlockSpec((1, 1, TILE), lambda i: (i, 0, 0)),
            full((D, D)), full((D, D)), full((1, D)), full((1, D)),
            full((D, D)), full((1, D)),
            full((D, D)), full((1, D)), full((1, D)),
            full((D, D)), full((1, D)),
            full((1, D)), full((1, D)),
        ],
        out_specs=pl.BlockSpec((TILE, D), lambda i: (i, 0)),
        out_shape=jax.ShapeDtypeStruct((N, D), jnp.float32),
        compiler_params=pltpu.CompilerParams(
            dimension_semantics=("arbitrary",)),
    )(t, o, m, osc,
      w1t, w1o, v1, u1, w2, c2,
      w3g, v3, u3, w4, c4,
      ln_g.reshape(1, D), ln_b.reshape(1, D))
    return out.reshape(B, S, D)


# speedup vs baseline: 2.9821x; 1.0057x over previous
import jax
import jax.numpy as jnp
from jax.experimental import pallas as pl
from jax.experimental.pallas import tpu as pltpu

D = 768
TILE = 1024

_INV_SQRT2 = 0.7071067811865476


def _gelu_exact(x):
    return 0.5 * x * (1.0 + jax.lax.erf(x * _INV_SQRT2))


def _body(t_ref, o_ref, ms_ref,
          w1t_ref, w1o_ref, v1_ref, u1_ref, w2_ref, c2_ref,
          w3_ref, v3_ref, u3_ref, w4_ref, c4_ref,
          gf_ref, bf_ref,
          out_ref):
    bf = jnp.bfloat16
    t = t_ref[...]
    o = o_ref[...]
    m = ms_ref[:, 0:1]
    osc = ms_ref[:, 1:2]

    st = jnp.sum(t, axis=1, keepdims=True)
    qt = jnp.sum(t * t, axis=1, keepdims=True)
    so = jnp.sum(o, axis=1, keepdims=True)
    qo = jnp.sum(o * o, axis=1, keepdims=True)
    mu = (st + osc * so) / (2 * D)
    var = (qt + osc * osc * qo) / (2 * D) - mu * mu
    inv = jax.lax.rsqrt(var + 1e-5)

    p = (jnp.dot(t.astype(bf), w1t_ref[...], preferred_element_type=jnp.float32)
         + osc * jnp.dot(o.astype(bf), w1o_ref[...],
                         preferred_element_type=jnp.float32))
    h = inv * (p - mu * v1_ref[0, :]) + u1_ref[0, :]
    h = _gelu_exact(h.astype(bf))
    x1 = (jnp.dot(h, w2_ref[...], preferred_element_type=jnp.float32)
          + c2_ref[0, :] + t)

    s1 = jnp.sum(x1, axis=1, keepdims=True)
    q1 = jnp.sum(x1 * x1, axis=1, keepdims=True)
    mu2 = s1 / D
    inv2 = jax.lax.rsqrt(q1 / D - mu2 * mu2 + 1e-5)
    h2 = (inv2 * (jnp.dot(x1.astype(bf), w3_ref[...],
                          preferred_element_type=jnp.float32)
                  - mu2 * v3_ref[0, :]) + u3_ref[0, :])
    h2 = _gelu_exact(h2.astype(bf))
    x2 = (jnp.dot(h2, w4_ref[...], preferred_element_type=jnp.float32)
          + c4_ref[0, :] + x1)

    mu3 = jnp.mean(x2, axis=1, keepdims=True)
    var3 = jnp.mean(x2 * x2, axis=1, keepdims=True) - mu3 * mu3
    y = (x2 - mu3) * jax.lax.rsqrt(var3 + 1e-5) * gf_ref[0, :] + bf_ref[0, :]

    out_ref[...] = jnp.where(m > 0, y, t)


def kernel(text_embeds, object_embeds, image_token_mask, num_objects,
           mlp1_ln_g, mlp1_ln_b, mlp1_fc1_w, mlp1_fc1_b, mlp1_fc2_w, mlp1_fc2_b,
           mlp2_ln_g, mlp2_ln_b, mlp2_fc1_w, mlp2_fc1_b, mlp2_fc2_w, mlp2_fc2_b,
           ln_g, ln_b):
    B, S, _ = text_embeds.shape
    N = B * S
    nb = N // TILE
    bf = jnp.bfloat16
    f32 = jnp.float32

    t = text_embeds.reshape(N, D)
    o = object_embeds.reshape(N, D)
    ms = jnp.stack(
        [image_token_mask.reshape(N).astype(f32),
         jnp.repeat((num_objects > 0).astype(f32), S)], axis=1)

    w1 = mlp1_fc1_w.T
    w1g = mlp1_ln_g[:, None] * w1
    w1t = w1g[:D, :].astype(bf)
    w1o = w1g[D:, :].astype(bf)
    v1 = (mlp1_ln_g @ w1).reshape(1, D)
    u1 = (mlp1_ln_b @ w1 + mlp1_fc1_b).reshape(1, D)
    w2 = mlp1_fc2_w.T.astype(bf)
    c2 = mlp1_fc2_b.reshape(1, D)
    w3 = mlp2_fc1_w.T
    w3g = (mlp2_ln_g[:, None] * w3).astype(bf)
    v3 = (mlp2_ln_g @ w3).reshape(1, D)
    u3 = (mlp2_ln_b @ w3 + mlp2_fc1_b).reshape(1, D)
    w4 = mlp2_fc2_w.T.astype(bf)
    c4 = mlp2_fc2_b.reshape(1, D)

    full = lambda shape: pl.BlockSpec(shape, lambda i: (0,) * len(shape))
    out = pl.pallas_call(
        _body,
        grid=(nb,),
        in_specs=[
            pl.BlockSpec((TILE, D), lambda i: (i, 0)),
            pl.BlockSpec((TILE, D), lambda i: (i, 0)),
            pl.BlockSpec((TILE, 2), lambda i: (i, 0)),
            full((D, D)), full((D, D)), full((1, D)), full((1, D)),
            full((D, D)), full((1, D)),
            full((D, D)), full((1, D)), full((1, D)),
            full((D, D)), full((1, D)),
            full((1, D)), full((1, D)),
        ],
        out_specs=pl.BlockSpec((TILE, D), lambda i: (i, 0)),
        out_shape=jax.ShapeDtypeStruct((N, D), jnp.float32),
        compiler_params=pltpu.CompilerParams(
            dimension_semantics=("arbitrary",)),
    )(t, o, ms,
      w1t, w1o, v1, u1, w2, c2,
      w3g, v3, u3, w4, c4,
      ln_g.reshape(1, D), ln_b.reshape(1, D))
    return out.reshape(B, S, D)


# speedup vs baseline: 3.0865x; 1.0350x over previous
import jax
import jax.numpy as jnp
from jax.experimental import pallas as pl
from jax.experimental.pallas import tpu as pltpu

D = 768
TILE = 1024

_INV_SQRT2 = 0.7071067811865476
_DNT = (((1,), (1,)), ((), ()))


def _gelu_exact(x):
    return 0.5 * x * (1.0 + jax.lax.erf(x * _INV_SQRT2))


def _mmt(x, w):
    return jax.lax.dot_general(x, w, _DNT, preferred_element_type=jnp.float32)


def _body(t_ref, o_ref, ms_ref,
          w1_ref, v1_ref, u1_ref, w2_ref, c2_ref,
          w3_ref, v3_ref, u3_ref, w4_ref, c4_ref,
          gf_ref, bf_ref,
          out_ref):
    bf = jnp.bfloat16
    t = t_ref[...]
    o = o_ref[...]
    m = ms_ref[:, 0:1]
    osc = ms_ref[:, 1:2]

    st = jnp.sum(t, axis=1, keepdims=True)
    qt = jnp.sum(t * t, axis=1, keepdims=True)
    so = jnp.sum(o, axis=1, keepdims=True)
    qo = jnp.sum(o * o, axis=1, keepdims=True)
    mu = (st + osc * so) / (2 * D)
    var = (qt + osc * osc * qo) / (2 * D) - mu * mu
    inv = jax.lax.rsqrt(var + 1e-5)

    p = (_mmt(t.astype(bf), w1_ref[:, :D])
         + osc * _mmt(o.astype(bf), w1_ref[:, D:]))
    h = inv * (p - mu * v1_ref[0, :]) + u1_ref[0, :]
    h = _gelu_exact(h.astype(bf))
    x1 = _mmt(h, w2_ref[...]) + c2_ref[0, :] + t

    s1 = jnp.sum(x1, axis=1, keepdims=True)
    q1 = jnp.sum(x1 * x1, axis=1, keepdims=True)
    mu2 = s1 / D
    inv2 = jax.lax.rsqrt(q1 / D - mu2 * mu2 + 1e-5)
    h2 = inv2 * (_mmt(x1.astype(bf), w3_ref[...]) - mu2 * v3_ref[0, :]) + u3_ref[0, :]
    h2 = _gelu_exact(h2.astype(bf))
    x2 = _mmt(h2, w4_ref[...]) + c4_ref[0, :] + x1

    mu3 = jnp.mean(x2, axis=1, keepdims=True)
    var3 = jnp.mean(x2 * x2, axis=1, keepdims=True) - mu3 * mu3
    y = (x2 - mu3) * jax.lax.rsqrt(var3 + 1e-5) * gf_ref[0, :] + bf_ref[0, :]

    out_ref[...] = jnp.where(m > 0, y, t)


def kernel(text_embeds, object_embeds, image_token_mask, num_objects,
           mlp1_ln_g, mlp1_ln_b, mlp1_fc1_w, mlp1_fc1_b, mlp1_fc2_w, mlp1_fc2_b,
           mlp2_ln_g, mlp2_ln_b, mlp2_fc1_w, mlp2_fc1_b, mlp2_fc2_w, mlp2_fc2_b,
           ln_g, ln_b):
    B, S, _ = text_embeds.shape
    N = B * S
    nb = N // TILE
    bf = jnp.bfloat16
    f32 = jnp.float32

    t = text_embeds.reshape(N, D)
    o = object_embeds.reshape(N, D)
    ms = jnp.stack(
        [image_token_mask.reshape(N).astype(f32),
         jnp.repeat((num_objects > 0).astype(f32), S)], axis=1)

    w1 = (mlp1_fc1_w * mlp1_ln_g[None, :]).astype(bf)
    v1 = (mlp1_fc1_w @ mlp1_ln_g).reshape(1, D)
    u1 = (mlp1_fc1_w @ mlp1_ln_b + mlp1_fc1_b).reshape(1, D)
    w2 = mlp1_fc2_w.astype(bf)
    c2 = mlp1_fc2_b.reshape(1, D)
    w3 = (mlp2_fc1_w * mlp2_ln_g[None, :]).astype(bf)
    v3 = (mlp2_fc1_w @ mlp2_ln_g).reshape(1, D)
    u3 = (mlp2_fc1_w @ mlp2_ln_b + mlp2_fc1_b).reshape(1, D)
    w4 = mlp2_fc2_w.astype(bf)
    c4 = mlp2_fc2_b.reshape(1, D)

    full = lambda shape: pl.BlockSpec(shape, lambda i: (0,) * len(shape))
    out = pl.pallas_call(
        _body,
        grid=(nb,),
        in_specs=[
            pl.BlockSpec((TILE, D), lambda i: (i, 0)),
            pl.BlockSpec((TILE, D), lambda i: (i, 0)),
            pl.BlockSpec((TILE, 2), lambda i: (i, 0)),
            full((D, 2 * D)), full((1, D)), full((1, D)),
            full((D, D)), full((1, D)),
            full((D, D)), full((1, D)), full((1, D)),
            full((D, D)), full((1, D)),
            full((1, D)), full((1, D)),
        ],
        out_specs=pl.BlockSpec((TILE, D), lambda i: (i, 0)),
        out_shape=jax.ShapeDtypeStruct((N, D), jnp.float32),
        compiler_params=pltpu.CompilerParams(
            dimension_semantics=("arbitrary",)),
    )(t, o, ms,
      w1, v1, u1, w2, c2,
      w3, v3, u3, w4, c4,
      ln_g.reshape(1, D), ln_b.reshape(1, D))
    return out.reshape(B, S, D)


# speedup vs baseline: 3.1495x; 1.0204x over previous
import jax
import jax.numpy as jnp
from jax.experimental import pallas as pl
from jax.experimental.pallas import tpu as pltpu

D = 768
TILE = 1024

_INV_SQRT2 = 0.7071067811865476
_DNT = (((1,), (1,)), ((), ()))


def _gelu_exact(x):
    return 0.5 * x * (1.0 + jax.lax.erf(x * _INV_SQRT2))


def _mmt(x, w):
    return jax.lax.dot_general(x, w, _DNT, preferred_element_type=jnp.float32)


def _body(t_ref, o_ref, ms_ref,
          w1_ref, w2_ref, w3_ref, w4_ref,
          g1_ref, gb1_ref, b1p_ref, c2_ref,
          g2_ref, gb2_ref, b2p_ref, c4_ref,
          gf_ref, bf_ref,
          out_ref,
          w1s_ref, w2s_ref, w3s_ref, w4s_ref, vu1_ref, vu3_ref):
    bf = jnp.bfloat16
    i = pl.program_id(0)

    @pl.when(i == 0)
    def _prep():
        w1s_ref[...] = (w1_ref[...] * g1_ref[0, :]).astype(bf)
        w2s_ref[...] = w2_ref[...].astype(bf)
        w3s_ref[...] = (w3_ref[...] * g2_ref[0, :]).astype(bf)
        w4s_ref[...] = w4_ref[...].astype(bf)
        vu1_ref[...] = _mmt(gb1_ref[...], w1_ref[...]) + b1p_ref[...]
        vu3_ref[...] = _mmt(gb2_ref[...], w3_ref[...]) + b2p_ref[...]

    t = t_ref[...]
    o = o_ref[...]
    m = ms_ref[:, 0:1]
    osc = ms_ref[:, 1:2]

    st = jnp.sum(t, axis=1, keepdims=True)
    qt = jnp.sum(t * t, axis=1, keepdims=True)
    so = jnp.sum(o, axis=1, keepdims=True)
    qo = jnp.sum(o * o, axis=1, keepdims=True)
    mu = (st + osc * so) / (2 * D)
    var = (qt + osc * osc * qo) / (2 * D) - mu * mu
    inv = jax.lax.rsqrt(var + 1e-5)

    p = (_mmt(t.astype(bf), w1s_ref[:, :D])
         + osc * _mmt(o.astype(bf), w1s_ref[:, D:]))
    h = inv * (p - mu * vu1_ref[0:1, :]) + vu1_ref[1:2, :]
    h = _gelu_exact(h.astype(bf))
    x1 = _mmt(h, w2s_ref[...]) + c2_ref[0, :] + t

    s1 = jnp.sum(x1, axis=1, keepdims=True)
    q1 = jnp.sum(x1 * x1, axis=1, keepdims=True)
    mu2 = s1 / D
    inv2 = jax.lax.rsqrt(q1 / D - mu2 * mu2 + 1e-5)
    h2 = inv2 * (_mmt(x1.astype(bf), w3s_ref[...]) - mu2 * vu3_ref[0:1, :]) \
        + vu3_ref[1:2, :]
    h2 = _gelu_exact(h2.astype(bf))
    x2 = _mmt(h2, w4s_ref[...]) + c4_ref[0, :] + x1

    mu3 = jnp.mean(x2, axis=1, keepdims=True)
    var3 = jnp.mean(x2 * x2, axis=1, keepdims=True) - mu3 * mu3
    y = (x2 - mu3) * jax.lax.rsqrt(var3 + 1e-5) * gf_ref[0, :] + bf_ref[0, :]

    out_ref[...] = jnp.where(m > 0, y, t)


def kernel(text_embeds, object_embeds, image_token_mask, num_objects,
           mlp1_ln_g, mlp1_ln_b, mlp1_fc1_w, mlp1_fc1_b, mlp1_fc2_w, mlp1_fc2_b,
           mlp2_ln_g, mlp2_ln_b, mlp2_fc1_w, mlp2_fc1_b, mlp2_fc2_w, mlp2_fc2_b,
           ln_g, ln_b):
    B, S, _ = text_embeds.shape
    N = B * S
    nb = N // TILE
    bf = jnp.bfloat16
    f32 = jnp.float32

    t = text_embeds.reshape(N, D)
    o = object_embeds.reshape(N, D)
    ms = jnp.stack(
        [image_token_mask.reshape(N).astype(f32),
         jnp.repeat((num_objects > 0).astype(f32), S)], axis=1)

    gb1 = jnp.zeros((8, 2 * D), f32).at[0].set(mlp1_ln_g).at[1].set(mlp1_ln_b)
    b1p = jnp.zeros((8, D), f32).at[1].set(mlp1_fc1_b)
    gb2 = jnp.zeros((8, D), f32).at[0].set(mlp2_ln_g).at[1].set(mlp2_ln_b)
    b2p = jnp.zeros((8, D), f32).at[1].set(mlp2_fc1_b)

    full = lambda shape: pl.BlockSpec(shape, lambda i: (0,) * len(shape))
    out = pl.pallas_call(
        _body,
        grid=(nb,),
        in_specs=[
            pl.BlockSpec((TILE, D), lambda i: (i, 0)),
            pl.BlockSpec((TILE, D), lambda i: (i, 0)),
            pl.BlockSpec((TILE, 2), lambda i: (i, 0)),
            full((D, 2 * D)), full((D, D)), full((D, D)), full((D, D)),
            full((1, 2 * D)), full((8, 2 * D)), full((8, D)), full((1, D)),
            full((1, D)), full((8, D)), full((8, D)), full((1, D)),
            full((1, D)), full((1, D)),
        ],
        out_specs=pl.BlockSpec((TILE, D), lambda i: (i, 0)),
        out_shape=jax.ShapeDtypeStruct((N, D), jnp.float32),
        scratch_shapes=[
            pltpu.VMEM((D, 2 * D), bf), pltpu.VMEM((D, D), bf),
            pltpu.VMEM((D, D), bf), pltpu.VMEM((D, D), bf),
            pltpu.VMEM((8, D), f32), pltpu.VMEM((8, D), f32),
        ],
        compiler_params=pltpu.CompilerParams(
            dimension_semantics=("arbitrary",)),
    )(t, o, ms,
      mlp1_fc1_w, mlp1_fc2_w, mlp2_fc1_w, mlp2_fc2_w,
      mlp1_ln_g.reshape(1, 2 * D), gb1, b1p, mlp1_fc2_b.reshape(1, D),
      mlp2_ln_g.reshape(1, D), gb2, b2p, mlp2_fc2_b.reshape(1, D),
      ln_g.reshape(1, D), ln_b.reshape(1, D))
    return out.reshape(B, S, D)
